# baseline (speedup 1.0000x reference)
import functools

import jax
import jax.numpy as jnp
from jax import lax
from jax.experimental import pallas as pl
from jax.experimental.pallas import tpu as pltpu
from jax.experimental.pallas import tpu_sc as plsc

N = 10000
E = 320000
D = 128
M = 2000
NC = 2
NS = 16
NW = NC * NS

ECH = 64
ENCH = 160
EQ = 40
EPW = ENCH * ECH
EPAD = NW * EPW
NACC = 10240
RPT = NACC // NS

XPAD = 10240
XCH = 80
MP = 2048
MCH = 64

_EPS = 1e-5


def _sc_mesh():
    return plsc.VectorSubcoreMesh(
        core_axis_name="c", subcore_axis_name="s", num_cores=NC, num_subcores=NS
    )


@functools.cache
def _get_agg():
    @functools.partial(
        pl.kernel,
        out_type=jax.ShapeDtypeStruct((NC, NACC, D), jnp.float32),
        mesh=_sc_mesh(),
        scratch_types=[
            pltpu.VMEM((EQ, ECH), jnp.int32),
            pltpu.VMEM((EQ, ECH), jnp.int32),
            pltpu.VMEM((4, ECH, D), jnp.float32),
            pltpu.VMEM_SHARED((NACC, D), jnp.float32),
            pltpu.SemaphoreType.DMA((4,)),
            pltpu.SemaphoreType.DMA((4,)),
        ],
    )
    def _agg(h_hbm, src_hbm, dst_hbm, out_hbm, sidx, didx, rows, acc, sg, ss):
        c = lax.axis_index("c")
        s = lax.axis_index("s")
        wid = c * NS + s

        pltpu.make_async_copy(
            src_hbm.at[pl.ds(wid * ENCH, EQ)], sidx, sg.at[0]
        ).start()
        pltpu.make_async_copy(
            dst_hbm.at[pl.ds(wid * ENCH, EQ)], didx, sg.at[1]
        ).start()

        zsrc = rows.at[3]

        def zrow(j, carry):
            def zcol(k, carry2):
                rows[3, j, pl.ds(k * 16, 16)] = jnp.zeros((16,), jnp.float32)
                return carry2
            return lax.fori_loop(0, D // 16, zcol, carry)

        lax.fori_loop(0, ECH, zrow, 0)
        row0 = s * RPT
        for t in range(RPT // ECH):
            pltpu.sync_copy(zsrc, acc.at[pl.ds(row0 + t * ECH, ECH)])
        plsc.subcore_barrier()

        bufs = [rows.at[k] for k in range(4)]

        def gath(i, k):
            return pltpu.make_async_copy(h_hbm.at[sidx.at[i]], bufs[k], sg.at[k])

        def scat_start(i, k):
            pltpu.async_copy(bufs[k], acc.at[didx.at[i]], ss.at[k], add=True)

        def scat_wait(i, k):
            pltpu.make_async_copy(bufs[k], acc.at[didx.at[i]], ss.at[k]).wait()

        for q in range(4):
            base = wid * ENCH + q * EQ
            if q == 0:
                pltpu.make_async_copy(
                    src_hbm.at[pl.ds(base, EQ)], sidx, sg.at[0]
                ).wait()
                pltpu.make_async_copy(
                    dst_hbm.at[pl.ds(base, EQ)], didx, sg.at[1]
                ).wait()
            else:
                pltpu.sync_copy(src_hbm.at[pl.ds(base, EQ)], sidx)
                pltpu.sync_copy(dst_hbm.at[pl.ds(base, EQ)], didx)
            gath(0, 0).start()
            gath(1, 1).start()

            def quad(g, carry):
                i0 = 4 * g
                for k in range(4):
                    i = i0 + k
                    j = (k + 2) % 4
                    gath(i, k).wait()
                    scat_start(i, k)

                    @pl.when(i >= 2)
                    def _():
                        scat_wait(i - 2, j)

                    @pl.when(i + 2 < EQ)
                    def _():
                        gath(i + 2, j).start()

                return carry

            lax.fori_loop(0, EQ // 4, quad, 0)
            scat_wait(EQ - 2, (EQ - 2) % 4)
            scat_wait(EQ - 1, (EQ - 1) % 4)
        plsc.subcore_barrier()
        pltpu.sync_copy(acc.at[pl.ds(row0, RPT)], out_hbm.at[c, pl.ds(row0, RPT)])

    return _agg


@functools.cache
def _make_gather(n_idx, ch, n_out):
    npw = n_idx // NW
    kc = npw // ch

    @functools.partial(
        pl.kernel,
        out_type=jax.ShapeDtypeStruct((n_out, D), jnp.float32),
        mesh=_sc_mesh(),
        scratch_types=[
            pltpu.VMEM((npw,), jnp.int32),
            pltpu.VMEM((ch, D), jnp.float32),
            pltpu.SemaphoreType.DMA,
        ],
    )
    def g(tab_hbm, idx_hbm, out_hbm, idxv, rows, sem):
        c = lax.axis_index("c")
        s = lax.axis_index("s")
        wid = c * NS + s
        pltpu.sync_copy(idx_hbm.at[pl.ds(wid * npw, npw)], idxv)

        def chunk(j, carry):
            row0 = wid * npw + j * ch

            @pl.when(row0 < n_out)
            def _():
                pltpu.async_copy(
                    tab_hbm.at[idxv.at[pl.ds(j * ch, ch)]], rows, sem
                ).wait()
                pltpu.sync_copy(rows, out_hbm.at[pl.ds(row0, ch)])

            return carry

        lax.fori_loop(0, kc, chunk, 0)

    return g


@functools.cache
def _get_mask_gather():
    npw = MP // NW
    tail = M - (NW - 1) * npw

    @functools.partial(
        pl.kernel,
        out_type=(
            jax.ShapeDtypeStruct((M, D), jnp.float32),
            jax.ShapeDtypeStruct((M, D), jnp.float32),
        ),
        mesh=_sc_mesh(),
        scratch_types=[
            pltpu.VMEM((npw,), jnp.int32),
            pltpu.VMEM((npw, D), jnp.float32),
            pltpu.SemaphoreType.DMA,
        ],
    )
    def g(ta_hbm, tb_hbm, idx_hbm, oa_hbm, ob_hbm, idxv, rows, sem):
        c = lax.axis_index("c")
        s = lax.axis_index("s")
        wid = c * NS + s
        row0 = wid * npw
        pltpu.sync_copy(idx_hbm.at[pl.ds(row0, npw)], idxv)
        for tab, out in ((ta_hbm, oa_hbm), (tb_hbm, ob_hbm)):
            pltpu.async_copy(tab.at[idxv], rows, sem).wait()

            @pl.when(row0 + npw <= M)
            def _():
                pltpu.sync_copy(rows, out.at[pl.ds(row0, npw)])

            @pl.when(row0 + npw > M)
            def _():
                pltpu.sync_copy(
                    rows.at[pl.ds(0, tail)], out.at[pl.ds(row0, tail)]
                )

    return g


def _bn_prelu(o, g, be, a):
    m = jnp.mean(o, axis=0, keepdims=True)
    v = jnp.mean((o - m) ** 2, axis=0, keepdims=True)
    o = g * (o - m) * lax.rsqrt(v + _EPS) + be
    return jnp.where(o >= 0, o, a * o)


def _mlp_body(h_ref, agg_ref, w1t, b1, g1, be1, a1, w2t, b2, g2, be2, a2, out_ref):
    h = h_ref[...]
    o = h + agg_ref[0, :N] + agg_ref[1, :N]
    o = jnp.dot(o, w1t[...], preferred_element_type=jnp.float32) + b1[...]
    o = _bn_prelu(o, g1[...], be1[...], a1[0, 0])
    o = jnp.dot(o, w2t[...], preferred_element_type=jnp.float32) + b2[...]
    o = _bn_prelu(o, g2[...], be2[...], a2[0, 0])
    out_ref[...] = o + h


_mlp = pl.pallas_call(
    _mlp_body, out_shape=jax.ShapeDtypeStruct((N, D), jnp.float32)
)


def _stats_body(h_ref, agg_ref, w1t, b1, g1, be1, a1, a_out, mean_out, c_out):
    h = h_ref[...]
    o = h + agg_ref[0, :N] + agg_ref[1, :N]
    o = jnp.dot(o, w1t[...], preferred_element_type=jnp.float32) + b1[...]
    a = _bn_prelu(o, g1[...], be1[...], a1[0, 0])
    a_out[...] = a
    mean_out[...] = jnp.mean(a, axis=0, keepdims=True)
    c_out[...] = lax.dot_general(
        a, a, (((0,), (0,)), ((), ())), preferred_element_type=jnp.float32
    ) * (1.0 / N)


_stats = pl.pallas_call(
    _stats_body,
    out_shape=[
        jax.ShapeDtypeStruct((N, D), jnp.float32),
        jax.ShapeDtypeStruct((1, D), jnp.float32),
        jax.ShapeDtypeStruct((D, D), jnp.float32),
    ],
)


def _final_body(am, hm, mean, cmat, w2t, b2, g2, be2, a2, rwt, out_ref):
    w2 = w2t[...]
    z = jnp.dot(am[...], w2, preferred_element_type=jnp.float32) + b2[...]
    mu = jnp.dot(mean[...], w2, preferred_element_type=jnp.float32) + b2[...]
    mvec = mean[...]
    cov = cmat[...] - lax.dot_general(
        mvec, mvec, (((0,), (0,)), ((), ())), preferred_element_type=jnp.float32
    )
    t = jnp.dot(cov, w2, preferred_element_type=jnp.float32)
    var = jnp.sum(w2 * t, axis=0, keepdims=True)
    y = g2[...] * (z - mu) * lax.rsqrt(var + _EPS) + be2[...]
    y = jnp.where(y >= 0, y, a2[0, 0] * y)
    logits = y + jnp.dot(hm[...], rwt[...], preferred_element_type=jnp.float32)
    mx = jnp.max(logits, axis=1, keepdims=True)
    lse = jnp.log(jnp.sum(jnp.exp(logits - mx), axis=1, keepdims=True)) + mx
    out_ref[...] = logits - lse


_final = pl.pallas_call(
    _final_body, out_shape=jax.ShapeDtypeStruct((M, 1000), jnp.float32)
)


def kernel(x, edge_index, mask_x_position, emb, params):
    pad_e = EPAD - E
    iota = jnp.arange(pad_e, dtype=jnp.int32)
    src = jnp.concatenate([edge_index[0], iota % N]).reshape(NW * ENCH, ECH)
    dst = jnp.concatenate(
        [edge_index[1], N + iota % (NACC - N)]
    ).reshape(NW * ENCH, ECH)
    xpad = jnp.concatenate([x, jnp.zeros((XPAD - N,), jnp.int32)])
    _agg = _get_agg()
    _emb_gather = _make_gather(XPAD, XCH, N)
    _mask_gather = _get_mask_gather()
    h = _emb_gather(emb, xpad)

    def layer_args(p):
        return (
            p["w1"].T, p["b1"][None], p["g1"][None], p["be1"][None],
            p["a1"].reshape(1, 1),
        )

    for li in range(4):
        p = params[li]
        agg = _agg(h, src, dst)
        h = _mlp(
            h, agg, *layer_args(p),
            p["w2"].T, p["b2"][None], p["g2"][None], p["be2"][None],
            p["a2"].reshape(1, 1),
        )

    p = params[4]
    agg = _agg(h, src, dst)
    a_full, mean, cmat = _stats(h, agg, *layer_args(p))
    mpad = jnp.concatenate([mask_x_position, jnp.zeros((MP - M,), jnp.int32)])
    am, hm = _mask_gather(a_full, h, mpad)
    return _final(
        am, hm, mean, cmat,
        p["w2"].T, p["b2"][None], p["g2"][None], p["be2"][None],
        p["a2"].reshape(1, 1), p["res_w"].T,
    )


# speedup vs baseline: 1.1401x; 1.1401x over previous
import functools

import jax
import jax.numpy as jnp
from jax import lax
from jax.experimental import pallas as pl
from jax.experimental.pallas import tpu as pltpu
from jax.experimental.pallas import tpu_sc as plsc

N = 10000
E = 320000
D = 128
M = 2000
NC = 2
NS = 16
NW = NC * NS

ECH = 128
ENCH = 80
EHALF = ENCH // 2
EPW = ENCH * ECH
EPAD = NW * EPW
NACC = 10240
RPT = NACC // NS

XPAD = 10240
XCH = 80
MP = 2048
MCH = 64

_EPS = 1e-5


def _sc_mesh():
    return plsc.VectorSubcoreMesh(
        core_axis_name="c", subcore_axis_name="s", num_cores=NC, num_subcores=NS
    )


@functools.cache
def _get_agg():
    @functools.partial(
        pl.kernel,
        out_type=jax.ShapeDtypeStruct((NC, NACC, D), jnp.float32),
        mesh=_sc_mesh(),
        scratch_types=[
            pltpu.VMEM((EHALF, ECH), jnp.int32),
            pltpu.VMEM((EHALF, ECH), jnp.int32),
            pltpu.VMEM((2, ECH, D), jnp.float32),
            pltpu.VMEM_SHARED((NACC, D), jnp.float32),
            pltpu.SemaphoreType.DMA,
            pltpu.SemaphoreType.DMA,
        ],
    )
    def _agg(h_hbm, src_hbm, dst_hbm, out_hbm, sidx, didx, rows, acc, sg0, sg1):
        c = lax.axis_index("c")
        s = lax.axis_index("s")
        wid = c * NS + s

        pltpu.make_async_copy(
            src_hbm.at[pl.ds(wid * ENCH, EHALF)], sidx, sg0
        ).start()
        pltpu.make_async_copy(
            dst_hbm.at[pl.ds(wid * ENCH, EHALF)], didx, sg1
        ).start()

        zsrc = rows.at[1]

        def zrow(j, carry):
            def zcol(k, carry2):
                rows[1, j, pl.ds(k * 16, 16)] = jnp.zeros((16,), jnp.float32)
                return carry2
            return lax.fori_loop(0, D // 16, zcol, carry)

        lax.fori_loop(0, ECH, zrow, 0)
        row0 = s * RPT
        for t in range(RPT // ECH):
            pltpu.sync_copy(zsrc, acc.at[pl.ds(row0 + t * ECH, ECH)])
        plsc.subcore_barrier()

        b0 = rows.at[0]
        b1 = rows.at[1]
        npairs = EHALF // 2

        def gath(i, buf, sem):
            return pltpu.make_async_copy(h_hbm.at[sidx.at[i]], buf, sem)

        for half in range(2):
            base = wid * ENCH + half * EHALF
            if half == 0:
                pltpu.make_async_copy(
                    src_hbm.at[pl.ds(base, EHALF)], sidx, sg0
                ).wait()
                pltpu.make_async_copy(
                    dst_hbm.at[pl.ds(base, EHALF)], didx, sg1
                ).wait()
            else:
                pltpu.sync_copy(src_hbm.at[pl.ds(base, EHALF)], sidx)
                pltpu.sync_copy(dst_hbm.at[pl.ds(base, EHALF)], didx)
            gath(0, b0, sg0).start()
            gath(1, b1, sg1).start()

            def pair(g, carry):
                i0 = 2 * g
                gath(i0, b0, sg0).wait()
                pltpu.sync_copy(b0, acc.at[didx.at[i0]], add=True)

                @pl.when(g < npairs - 1)
                def _():
                    gath(i0 + 2, b0, sg0).start()

                gath(i0 + 1, b1, sg1).wait()
                pltpu.sync_copy(b1, acc.at[didx.at[i0 + 1]], add=True)

                @pl.when(g < npairs - 1)
                def _():
                    gath(i0 + 3, b1, sg1).start()

                return carry

            lax.fori_loop(0, npairs, pair, 0)
        plsc.subcore_barrier()
        pltpu.sync_copy(acc.at[pl.ds(row0, RPT)], out_hbm.at[c, pl.ds(row0, RPT)])

    return _agg


@functools.cache
def _make_gather(n_idx, ch, n_out):
    npw = n_idx // NW
    kc = npw // ch

    @functools.partial(
        pl.kernel,
        out_type=jax.ShapeDtypeStruct((n_out, D), jnp.float32),
        mesh=_sc_mesh(),
        scratch_types=[
            pltpu.VMEM((npw,), jnp.int32),
            pltpu.VMEM((ch, D), jnp.float32),
            pltpu.SemaphoreType.DMA,
        ],
    )
    def g(tab_hbm, idx_hbm, out_hbm, idxv, rows, sem):
        c = lax.axis_index("c")
        s = lax.axis_index("s")
        wid = c * NS + s
        pltpu.sync_copy(idx_hbm.at[pl.ds(wid * npw, npw)], idxv)

        def chunk(j, carry):
            row0 = wid * npw + j * ch

            @pl.when(row0 < n_out)
            def _():
                pltpu.async_copy(
                    tab_hbm.at[idxv.at[pl.ds(j * ch, ch)]], rows, sem
                ).wait()
                pltpu.sync_copy(rows, out_hbm.at[pl.ds(row0, ch)])

            return carry

        lax.fori_loop(0, kc, chunk, 0)

    return g


@functools.cache
def _get_mask_gather():
    npw = MP // NW
    tail = M - (NW - 1) * npw

    @functools.partial(
        pl.kernel,
        out_type=(
            jax.ShapeDtypeStruct((M, D), jnp.float32),
            jax.ShapeDtypeStruct((M, D), jnp.float32),
        ),
        mesh=_sc_mesh(),
        scratch_types=[
            pltpu.VMEM((npw,), jnp.int32),
            pltpu.VMEM((npw, D), jnp.float32),
            pltpu.SemaphoreType.DMA,
        ],
    )
    def g(ta_hbm, tb_hbm, idx_hbm, oa_hbm, ob_hbm, idxv, rows, sem):
        c = lax.axis_index("c")
        s = lax.axis_index("s")
        wid = c * NS + s
        row0 = wid * npw
        pltpu.sync_copy(idx_hbm.at[pl.ds(row0, npw)], idxv)
        for tab, out in ((ta_hbm, oa_hbm), (tb_hbm, ob_hbm)):
            pltpu.async_copy(tab.at[idxv], rows, sem).wait()

            @pl.when(row0 + npw <= M)
            def _():
                pltpu.sync_copy(rows, out.at[pl.ds(row0, npw)])

            @pl.when(row0 + npw > M)
            def _():
                pltpu.sync_copy(
                    rows.at[pl.ds(0, tail)], out.at[pl.ds(row0, tail)]
                )

    return g


def _bn_prelu(o, g, be, a):
    m = jnp.mean(o, axis=0, keepdims=True)
    v = jnp.mean((o - m) ** 2, axis=0, keepdims=True)
    o = g * (o - m) * lax.rsqrt(v + _EPS) + be
    return jnp.where(o >= 0, o, a * o)


def _mlp_body(h_ref, agg_ref, w1t, b1, g1, be1, a1, w2t, b2, g2, be2, a2, out_ref):
    h = h_ref[...]
    o = h + agg_ref[0, :N] + agg_ref[1, :N]
    o = jnp.dot(o, w1t[...], preferred_element_type=jnp.float32) + b1[...]
    o = _bn_prelu(o, g1[...], be1[...], a1[0, 0])
    o = jnp.dot(o, w2t[...], preferred_element_type=jnp.float32) + b2[...]
    o = _bn_prelu(o, g2[...], be2[...], a2[0, 0])
    out_ref[...] = o + h


_mlp = pl.pallas_call(
    _mlp_body, out_shape=jax.ShapeDtypeStruct((N, D), jnp.float32)
)


def _stats_body(h_ref, agg_ref, w1t, b1, g1, be1, a1, a_out, mean_out, c_out):
    h = h_ref[...]
    o = h + agg_ref[0, :N] + agg_ref[1, :N]
    o = jnp.dot(o, w1t[...], preferred_element_type=jnp.float32) + b1[...]
    a = _bn_prelu(o, g1[...], be1[...], a1[0, 0])
    a_out[...] = a
    mean_out[...] = jnp.mean(a, axis=0, keepdims=True)
    c_out[...] = lax.dot_general(
        a, a, (((0,), (0,)), ((), ())), preferred_element_type=jnp.float32
    ) * (1.0 / N)


_stats = pl.pallas_call(
    _stats_body,
    out_shape=[
        jax.ShapeDtypeStruct((N, D), jnp.float32),
        jax.ShapeDtypeStruct((1, D), jnp.float32),
        jax.ShapeDtypeStruct((D, D), jnp.float32),
    ],
)


def _final_body(am, hm, mean, cmat, w2t, b2, g2, be2, a2, rwt, out_ref):
    w2 = w2t[...]
    z = jnp.dot(am[...], w2, preferred_element_type=jnp.float32) + b2[...]
    mu = jnp.dot(mean[...], w2, preferred_element_type=jnp.float32) + b2[...]
    mvec = mean[...]
    cov = cmat[...] - lax.dot_general(
        mvec, mvec, (((0,), (0,)), ((), ())), preferred_element_type=jnp.float32
    )
    t = jnp.dot(cov, w2, preferred_element_type=jnp.float32)
    var = jnp.sum(w2 * t, axis=0, keepdims=True)
    y = g2[...] * (z - mu) * lax.rsqrt(var + _EPS) + be2[...]
    y = jnp.where(y >= 0, y, a2[0, 0] * y)
    logits = y + jnp.dot(hm[...], rwt[...], preferred_element_type=jnp.float32)
    mx = jnp.max(logits, axis=1, keepdims=True)
    lse = jnp.log(jnp.sum(jnp.exp(logits - mx), axis=1, keepdims=True)) + mx
    out_ref[...] = logits - lse


_final = pl.pallas_call(
    _final_body, out_shape=jax.ShapeDtypeStruct((M, 1000), jnp.float32)
)


def kernel(x, edge_index, mask_x_position, emb, params):
    pad_e = EPAD - E
    iota = jnp.arange(pad_e, dtype=jnp.int32)
    src = jnp.concatenate([edge_index[0], iota % N]).reshape(NW * ENCH, ECH)
    dst = jnp.concatenate(
        [edge_index[1], N + iota % (NACC - N)]
    ).reshape(NW * ENCH, ECH)
    xpad = jnp.concatenate([x, jnp.zeros((XPAD - N,), jnp.int32)])
    _agg = _get_agg()
    _emb_gather = _make_gather(XPAD, XCH, N)
    _mask_gather = _get_mask_gather()
    h = _emb_gather(emb, xpad)

    def layer_args(p):
        return (
            p["w1"].T, p["b1"][None], p["g1"][None], p["be1"][None],
            p["a1"].reshape(1, 1),
        )

    for li in range(4):
        p = params[li]
        agg = _agg(h, src, dst)
        h = _mlp(
            h, agg, *layer_args(p),
            p["w2"].T, p["b2"][None], p["g2"][None], p["be2"][None],
            p["a2"].reshape(1, 1),
        )

    p = params[4]
    agg = _agg(h, src, dst)
    a_full, mean, cmat = _stats(h, agg, *layer_args(p))
    mpad = jnp.concatenate([mask_x_position, jnp.zeros((MP - M,), jnp.int32)])
    am, hm = _mask_gather(a_full, h, mpad)
    return _final(
        am, hm, mean, cmat,
        p["w2"].T, p["b2"][None], p["g2"][None], p["be2"][None],
        p["a2"].reshape(1, 1), p["res_w"].T,
    )


# speedup vs baseline: 1.1446x; 1.0039x over previous
import functools

import jax
import jax.numpy as jnp
from jax import lax
from jax.experimental import pallas as pl
from jax.experimental.pallas import tpu as pltpu
from jax.experimental.pallas import tpu_sc as plsc

N = 10000
E = 320000
D = 128
M = 2000
NC = 2
NS = 16
NW = NC * NS

ECH = 128
ENCH = 80
EHALF = ENCH // 2
EPW = ENCH * ECH
EPAD = NW * EPW
NACC = 10240
RPT = NACC // NS

XPAD = 10240
XCH = 80
MP = 2048
MCH = 64

_EPS = 1e-5


def _sc_mesh():
    return plsc.VectorSubcoreMesh(
        core_axis_name="c", subcore_axis_name="s", num_cores=NC, num_subcores=NS
    )


@functools.cache
def _get_agg():
    @functools.partial(
        pl.kernel,
        out_type=jax.ShapeDtypeStruct((NC, NACC, D), jnp.float32),
        mesh=_sc_mesh(),
        scratch_types=[
            pltpu.VMEM((EHALF, ECH), jnp.int32),
            pltpu.VMEM((EHALF, ECH), jnp.int32),
            pltpu.VMEM((2, ECH, D), jnp.float32),
            pltpu.VMEM_SHARED((NACC, D), jnp.float32),
            pltpu.SemaphoreType.DMA,
            pltpu.SemaphoreType.DMA,
        ],
    )
    def _agg(h_hbm, src_hbm, dst_hbm, out_hbm, sidx, didx, rows, acc, sg0, sg1):
        c = lax.axis_index("c")
        s = lax.axis_index("s")
        wid = c * NS + s

        pltpu.make_async_copy(
            src_hbm.at[pl.ds(wid * ENCH, EHALF)], sidx, sg0
        ).start()
        pltpu.make_async_copy(
            dst_hbm.at[pl.ds(wid * ENCH, EHALF)], didx, sg1
        ).start()

        zsrc = rows.at[1]

        def zrow(j, carry):
            def zcol(k, carry2):
                rows[1, j, pl.ds(k * 16, 16)] = jnp.zeros((16,), jnp.float32)
                return carry2
            return lax.fori_loop(0, D // 16, zcol, carry)

        lax.fori_loop(0, ECH, zrow, 0)
        row0 = s * RPT
        for t in range(RPT // ECH):
            pltpu.sync_copy(zsrc, acc.at[pl.ds(row0 + t * ECH, ECH)])

        b0 = rows.at[0]
        b1 = rows.at[1]
        npairs = EHALF // 2

        def gath(i, buf, sem):
            return pltpu.make_async_copy(h_hbm.at[sidx.at[i]], buf, sem)

        pltpu.make_async_copy(
            src_hbm.at[pl.ds(wid * ENCH, EHALF)], sidx, sg0
        ).wait()
        pltpu.make_async_copy(
            dst_hbm.at[pl.ds(wid * ENCH, EHALF)], didx, sg1
        ).wait()
        gath(0, b0, sg0).start()
        gath(1, b1, sg1).start()
        plsc.subcore_barrier()

        for half in range(2):
            base = wid * ENCH + half * EHALF
            if half == 1:
                pltpu.sync_copy(src_hbm.at[pl.ds(base, EHALF)], sidx)
                pltpu.sync_copy(dst_hbm.at[pl.ds(base, EHALF)], didx)
                gath(0, b0, sg0).start()
                gath(1, b1, sg1).start()

            def pair(g, carry):
                i0 = 2 * g
                gath(i0, b0, sg0).wait()
                pltpu.sync_copy(b0, acc.at[didx.at[i0]], add=True)

                @pl.when(g < npairs - 1)
                def _():
                    gath(i0 + 2, b0, sg0).start()

                gath(i0 + 1, b1, sg1).wait()
                pltpu.sync_copy(b1, acc.at[didx.at[i0 + 1]], add=True)

                @pl.when(g < npairs - 1)
                def _():
                    gath(i0 + 3, b1, sg1).start()

                return carry

            lax.fori_loop(0, npairs, pair, 0)
        plsc.subcore_barrier()
        pltpu.sync_copy(acc.at[pl.ds(row0, RPT)], out_hbm.at[c, pl.ds(row0, RPT)])

    return _agg


@functools.cache
def _make_gather(n_idx, ch, n_out):
    npw = n_idx // NW
    kc = npw // ch

    @functools.partial(
        pl.kernel,
        out_type=jax.ShapeDtypeStruct((n_out, D), jnp.float32),
        mesh=_sc_mesh(),
        scratch_types=[
            pltpu.VMEM((npw,), jnp.int32),
            pltpu.VMEM((ch, D), jnp.float32),
            pltpu.SemaphoreType.DMA,
        ],
    )
    def g(tab_hbm, idx_hbm, out_hbm, idxv, rows, sem):
        c = lax.axis_index("c")
        s = lax.axis_index("s")
        wid = c * NS + s
        pltpu.sync_copy(idx_hbm.at[pl.ds(wid * npw, npw)], idxv)

        def chunk(j, carry):
            row0 = wid * npw + j * ch

            @pl.when(row0 < n_out)
            def _():
                pltpu.async_copy(
                    tab_hbm.at[idxv.at[pl.ds(j * ch, ch)]], rows, sem
                ).wait()
                pltpu.sync_copy(rows, out_hbm.at[pl.ds(row0, ch)])

            return carry

        lax.fori_loop(0, kc, chunk, 0)

    return g


@functools.cache
def _get_mask_gather():
    npt = MP // NS
    tail = M - (NS - 1) * npt

    @functools.partial(
        pl.kernel,
        out_type=(
            jax.ShapeDtypeStruct((M, D), jnp.float32),
            jax.ShapeDtypeStruct((M, D), jnp.float32),
        ),
        mesh=_sc_mesh(),
        scratch_types=[
            pltpu.VMEM((npt,), jnp.int32),
            pltpu.VMEM((npt, D), jnp.float32),
            pltpu.SemaphoreType.DMA,
        ],
    )
    def g(ta_hbm, tb_hbm, idx_hbm, oa_hbm, ob_hbm, idxv, rows, sem):
        c = lax.axis_index("c")
        s = lax.axis_index("s")
        row0 = s * npt
        pltpu.sync_copy(idx_hbm.at[pl.ds(row0, npt)], idxv)
        for t, (tab, out) in enumerate(((ta_hbm, oa_hbm), (tb_hbm, ob_hbm))):
            @pl.when(c == t)
            def _():
                pltpu.async_copy(tab.at[idxv], rows, sem).wait()

                @pl.when(row0 + npt <= M)
                def _():
                    pltpu.sync_copy(rows, out.at[pl.ds(row0, npt)])

                @pl.when(row0 + npt > M)
                def _():
                    pltpu.sync_copy(
                        rows.at[pl.ds(0, tail)], out.at[pl.ds(row0, tail)]
                    )

    return g


def _bn_prelu(o, g, be, a):
    m = jnp.mean(o, axis=0, keepdims=True)
    v = jnp.mean((o - m) ** 2, axis=0, keepdims=True)
    o = g * (o - m) * lax.rsqrt(v + _EPS) + be
    return jnp.where(o >= 0, o, a * o)


def _mlp_body(h_ref, agg_ref, w1t, b1, g1, be1, a1, w2t, b2, g2, be2, a2, out_ref):
    h = h_ref[...]
    o = h + agg_ref[0, :N] + agg_ref[1, :N]
    o = jnp.dot(o, w1t[...], preferred_element_type=jnp.float32) + b1[...]
    o = _bn_prelu(o, g1[...], be1[...], a1[0, 0])
    o = jnp.dot(o, w2t[...], preferred_element_type=jnp.float32) + b2[...]
    o = _bn_prelu(o, g2[...], be2[...], a2[0, 0])
    out_ref[...] = o + h


_mlp = pl.pallas_call(
    _mlp_body, out_shape=jax.ShapeDtypeStruct((N, D), jnp.float32)
)


def _stats_body(h_ref, agg_ref, w1t, b1, g1, be1, a1, a_out, mean_out, c_out):
    h = h_ref[...]
    o = h + agg_ref[0, :N] + agg_ref[1, :N]
    o = jnp.dot(o, w1t[...], preferred_element_type=jnp.float32) + b1[...]
    a = _bn_prelu(o, g1[...], be1[...], a1[0, 0])
    a_out[...] = a
    mean_out[...] = jnp.mean(a, axis=0, keepdims=True)
    c_out[...] = lax.dot_general(
        a, a, (((0,), (0,)), ((), ())), preferred_element_type=jnp.float32
    ) * (1.0 / N)


_stats = pl.pallas_call(
    _stats_body,
    out_shape=[
        jax.ShapeDtypeStruct((N, D), jnp.float32),
        jax.ShapeDtypeStruct((1, D), jnp.float32),
        jax.ShapeDtypeStruct((D, D), jnp.float32),
    ],
)


def _final_body(am, hm, mean, cmat, w2t, b2, g2, be2, a2, rwt, out_ref):
    w2 = w2t[...]
    z = jnp.dot(am[...], w2, preferred_element_type=jnp.float32) + b2[...]
    mu = jnp.dot(mean[...], w2, preferred_element_type=jnp.float32) + b2[...]
    mvec = mean[...]
    cov = cmat[...] - lax.dot_general(
        mvec, mvec, (((0,), (0,)), ((), ())), preferred_element_type=jnp.float32
    )
    t = jnp.dot(cov, w2, preferred_element_type=jnp.float32)
    var = jnp.sum(w2 * t, axis=0, keepdims=True)
    y = g2[...] * (z - mu) * lax.rsqrt(var + _EPS) + be2[...]
    y = jnp.where(y >= 0, y, a2[0, 0] * y)
    logits = y + jnp.dot(hm[...], rwt[...], preferred_element_type=jnp.float32)
    mx = jnp.max(logits, axis=1, keepdims=True)
    lse = jnp.log(jnp.sum(jnp.exp(logits - mx), axis=1, keepdims=True)) + mx
    out_ref[...] = logits - lse


_final = pl.pallas_call(
    _final_body, out_shape=jax.ShapeDtypeStruct((M, 1000), jnp.float32)
)


def kernel(x, edge_index, mask_x_position, emb, params):
    pad_e = EPAD - E
    iota = jnp.arange(pad_e, dtype=jnp.int32)
    src = jnp.concatenate([edge_index[0], iota % N]).reshape(NW * ENCH, ECH)
    dst = jnp.concatenate(
        [edge_index[1], N + iota % (NACC - N)]
    ).reshape(NW * ENCH, ECH)
    xpad = jnp.concatenate([x, jnp.zeros((XPAD - N,), jnp.int32)])
    _agg = _get_agg()
    _emb_gather = _make_gather(XPAD, XCH, N)
    _mask_gather = _get_mask_gather()
    h = _emb_gather(emb, xpad)

    def layer_args(p):
        return (
            p["w1"].T, p["b1"][None], p["g1"][None], p["be1"][None],
            p["a1"].reshape(1, 1),
        )

    for li in range(4):
        p = params[li]
        agg = _agg(h, src, dst)
        h = _mlp(
            h, agg, *layer_args(p),
            p["w2"].T, p["b2"][None], p["g2"][None], p["be2"][None],
            p["a2"].reshape(1, 1),
        )

    p = params[4]
    agg = _agg(h, src, dst)
    a_full, mean, cmat = _stats(h, agg, *layer_args(p))
    mpad = jnp.concatenate([mask_x_position, jnp.zeros((MP - M,), jnp.int32)])
    am, hm = _mask_gather(a_full, h, mpad)
    return _final(
        am, hm, mean, cmat,
        p["w2"].T, p["b2"][None], p["g2"][None], p["be2"][None],
        p["a2"].reshape(1, 1), p["res_w"].T,
    )


# speedup vs baseline: 1.1459x; 1.0011x over previous
import functools

import jax
import jax.numpy as jnp
from jax import lax
from jax.experimental import pallas as pl
from jax.experimental.pallas import tpu as pltpu
from jax.experimental.pallas import tpu_sc as plsc

N = 10000
E = 320000
D = 128
M = 2000
NC = 2
NS = 16
NW = NC * NS

ECH = 128
ENCH = 80
EHALF = ENCH // 2
EPW = ENCH * ECH
EPAD = NW * EPW
NACC = 10240
RPT = NACC // NS

XPAD = 10240
XCH = 80
MP = 2048
MCH = 64

_EPS = 1e-5


def _sc_mesh():
    return plsc.VectorSubcoreMesh(
        core_axis_name="c", subcore_axis_name="s", num_cores=NC, num_subcores=NS
    )


@functools.cache
def _get_agg():
    @functools.partial(
        pl.kernel,
        out_type=jax.ShapeDtypeStruct((NC, NACC, D), jnp.float32),
        mesh=_sc_mesh(),
        scratch_types=[
            pltpu.VMEM((EHALF, ECH), jnp.int32),
            pltpu.VMEM((EHALF, ECH), jnp.int32),
            pltpu.VMEM((2, ECH, D), jnp.float32),
            pltpu.VMEM_SHARED((NACC, D), jnp.float32),
            pltpu.SemaphoreType.DMA,
            pltpu.SemaphoreType.DMA,
        ],
    )
    def _agg(h_hbm, src_hbm, dst_hbm, out_hbm, sidx, didx, rows, acc, sg0, sg1):
        c = lax.axis_index("c")
        s = lax.axis_index("s")
        wid = c * NS + s

        pltpu.make_async_copy(
            src_hbm.at[pl.ds(wid * ENCH, EHALF)], sidx, sg0
        ).start()
        pltpu.make_async_copy(
            dst_hbm.at[pl.ds(wid * ENCH, EHALF)], didx, sg1
        ).start()

        zsrc = rows.at[1]

        def zrow(j, carry):
            def zcol(k, carry2):
                rows[1, j, pl.ds(k * 16, 16)] = jnp.zeros((16,), jnp.float32)
                return carry2
            return lax.fori_loop(0, D // 16, zcol, carry)

        lax.fori_loop(0, ECH, zrow, 0)
        row0 = s * RPT
        for t in range(RPT // ECH):
            pltpu.sync_copy(zsrc, acc.at[pl.ds(row0 + t * ECH, ECH)])

        b0 = rows.at[0]
        b1 = rows.at[1]
        npairs = EHALF // 2

        def gath(i, buf, sem):
            return pltpu.make_async_copy(h_hbm.at[sidx.at[i]], buf, sem)

        pltpu.make_async_copy(
            src_hbm.at[pl.ds(wid * ENCH, EHALF)], sidx, sg0
        ).wait()
        pltpu.make_async_copy(
            dst_hbm.at[pl.ds(wid * ENCH, EHALF)], didx, sg1
        ).wait()
        gath(0, b0, sg0).start()
        gath(1, b1, sg1).start()
        plsc.subcore_barrier()

        for half in range(2):
            base = wid * ENCH + half * EHALF
            if half == 1:
                pltpu.sync_copy(src_hbm.at[pl.ds(base, EHALF)], sidx)
                pltpu.sync_copy(dst_hbm.at[pl.ds(base, EHALF)], didx)
                gath(0, b0, sg0).start()
                gath(1, b1, sg1).start()

            def pair(g, carry):
                i0 = 2 * g
                gath(i0, b0, sg0).wait()
                pltpu.sync_copy(b0, acc.at[didx.at[i0]], add=True)

                @pl.when(g < npairs - 1)
                def _():
                    gath(i0 + 2, b0, sg0).start()

                gath(i0 + 1, b1, sg1).wait()
                pltpu.sync_copy(b1, acc.at[didx.at[i0 + 1]], add=True)

                @pl.when(g < npairs - 1)
                def _():
                    gath(i0 + 3, b1, sg1).start()

                return carry

            lax.fori_loop(0, npairs, pair, 0)
        plsc.subcore_barrier()
        pltpu.sync_copy(acc.at[pl.ds(row0, RPT)], out_hbm.at[c, pl.ds(row0, RPT)])

    return _agg


@functools.cache
def _make_gather(n_idx, ch, n_out):
    npw = n_idx // NW
    kc = npw // ch

    @functools.partial(
        pl.kernel,
        out_type=jax.ShapeDtypeStruct((n_out, D), jnp.float32),
        mesh=_sc_mesh(),
        scratch_types=[
            pltpu.VMEM((npw,), jnp.int32),
            pltpu.VMEM((ch, D), jnp.float32),
            pltpu.SemaphoreType.DMA,
        ],
    )
    def g(tab_hbm, idx_hbm, out_hbm, idxv, rows, sem):
        c = lax.axis_index("c")
        s = lax.axis_index("s")
        wid = c * NS + s
        pltpu.sync_copy(idx_hbm.at[pl.ds(wid * npw, npw)], idxv)

        def chunk(j, carry):
            row0 = wid * npw + j * ch

            @pl.when(row0 < n_out)
            def _():
                pltpu.async_copy(
                    tab_hbm.at[idxv.at[pl.ds(j * ch, ch)]], rows, sem
                ).wait()
                pltpu.sync_copy(rows, out_hbm.at[pl.ds(row0, ch)])

            return carry

        lax.fori_loop(0, kc, chunk, 0)

    return g


@functools.cache
def _get_mask_gather():
    npt = MP // NS
    tail = M - (NS - 1) * npt

    @functools.partial(
        pl.kernel,
        out_type=(
            jax.ShapeDtypeStruct((M, D), jnp.float32),
            jax.ShapeDtypeStruct((M, D), jnp.float32),
        ),
        mesh=_sc_mesh(),
        scratch_types=[
            pltpu.VMEM((npt,), jnp.int32),
            pltpu.VMEM((npt, D), jnp.float32),
            pltpu.SemaphoreType.DMA,
        ],
    )
    def g(ta_hbm, tb_hbm, idx_hbm, oa_hbm, ob_hbm, idxv, rows, sem):
        c = lax.axis_index("c")
        s = lax.axis_index("s")
        row0 = s * npt
        pltpu.sync_copy(idx_hbm.at[pl.ds(row0, npt)], idxv)
        for t, (tab, out) in enumerate(((ta_hbm, oa_hbm), (tb_hbm, ob_hbm))):
            @pl.when(c == t)
            def _():
                pltpu.async_copy(tab.at[idxv], rows, sem).wait()

                @pl.when(row0 + npt <= M)
                def _():
                    pltpu.sync_copy(rows, out.at[pl.ds(row0, npt)])

                @pl.when(row0 + npt > M)
                def _():
                    pltpu.sync_copy(
                        rows.at[pl.ds(0, tail)], out.at[pl.ds(row0, tail)]
                    )

    return g


def _dot_t(x, w):
    return lax.dot_general(
        x, w, (((1,), (1,)), ((), ())), preferred_element_type=jnp.float32
    )


def _bn_prelu(o, g, be, a):
    m = jnp.mean(o, axis=0, keepdims=True)
    v = jnp.mean((o - m) ** 2, axis=0, keepdims=True)
    o = g * (o - m) * lax.rsqrt(v + _EPS) + be
    return jnp.where(o >= 0, o, a * o)


def _mlp_body(h_ref, agg_ref, w1, b1, g1, be1, a1, w2, b2, g2, be2, a2, out_ref):
    h = h_ref[...]
    o = h + agg_ref[0, :N] + agg_ref[1, :N]
    o = _dot_t(o, w1[...]) + b1[...]
    o = _bn_prelu(o, g1[...], be1[...], a1[0, 0])
    o = _dot_t(o, w2[...]) + b2[...]
    o = _bn_prelu(o, g2[...], be2[...], a2[0, 0])
    out_ref[...] = o + h


_mlp = pl.pallas_call(
    _mlp_body, out_shape=jax.ShapeDtypeStruct((N, D), jnp.float32)
)


def _stats_body(h_ref, agg_ref, w1, b1, g1, be1, a1, a_out, mean_out, c_out):
    h = h_ref[...]
    o = h + agg_ref[0, :N] + agg_ref[1, :N]
    o = _dot_t(o, w1[...]) + b1[...]
    a = _bn_prelu(o, g1[...], be1[...], a1[0, 0])
    a_out[...] = a
    mean_out[...] = jnp.mean(a, axis=0, keepdims=True)
    c_out[...] = lax.dot_general(
        a, a, (((0,), (0,)), ((), ())), preferred_element_type=jnp.float32
    ) * (1.0 / N)


_stats = pl.pallas_call(
    _stats_body,
    out_shape=[
        jax.ShapeDtypeStruct((N, D), jnp.float32),
        jax.ShapeDtypeStruct((1, D), jnp.float32),
        jax.ShapeDtypeStruct((D, D), jnp.float32),
    ],
)


def _final_body(am, hm, mean, cmat, w2_ref, b2, g2, be2, a2, rw, out_ref):
    w2 = w2_ref[...]
    z = _dot_t(am[...], w2) + b2[...]
    mu = _dot_t(mean[...], w2) + b2[...]
    mvec = mean[...]
    cov = cmat[...] - lax.dot_general(
        mvec, mvec, (((0,), (0,)), ((), ())), preferred_element_type=jnp.float32
    )
    t = _dot_t(w2, cov)
    var = jnp.sum(w2 * t, axis=1)[None]
    y = g2[...] * (z - mu) * lax.rsqrt(var + _EPS) + be2[...]
    y = jnp.where(y >= 0, y, a2[0, 0] * y)
    logits = y + _dot_t(hm[...], rw[...])
    mx = jnp.max(logits, axis=1, keepdims=True)
    lse = jnp.log(jnp.sum(jnp.exp(logits - mx), axis=1, keepdims=True)) + mx
    out_ref[...] = logits - lse


_final = pl.pallas_call(
    _final_body, out_shape=jax.ShapeDtypeStruct((M, 1000), jnp.float32)
)


def kernel(x, edge_index, mask_x_position, emb, params):
    pad_e = EPAD - E
    iota = jnp.arange(pad_e, dtype=jnp.int32)
    src = jnp.concatenate([edge_index[0], iota % N]).reshape(NW * ENCH, ECH)
    dst = jnp.concatenate(
        [edge_index[1], N + iota % (NACC - N)]
    ).reshape(NW * ENCH, ECH)
    xpad = jnp.concatenate([x, jnp.zeros((XPAD - N,), jnp.int32)])
    _agg = _get_agg()
    _emb_gather = _make_gather(XPAD, XCH, N)
    _mask_gather = _get_mask_gather()
    h = _emb_gather(emb, xpad)

    def layer_args(p):
        return (
            p["w1"], p["b1"][None], p["g1"][None], p["be1"][None],
            p["a1"].reshape(1, 1),
        )

    for li in range(4):
        p = params[li]
        agg = _agg(h, src, dst)
        h = _mlp(
            h, agg, *layer_args(p),
            p["w2"], p["b2"][None], p["g2"][None], p["be2"][None],
            p["a2"].reshape(1, 1),
        )

    p = params[4]
    agg = _agg(h, src, dst)
    a_full, mean, cmat = _stats(h, agg, *layer_args(p))
    mpad = jnp.concatenate([mask_x_position, jnp.zeros((MP - M,), jnp.int32)])
    am, hm = _mask_gather(a_full, h, mpad)
    return _final(
        am, hm, mean, cmat,
        p["w2"], p["b2"][None], p["g2"][None], p["be2"][None],
        p["a2"].reshape(1, 1), p["res_w"],
    )
